# baseline (speedup 1.0000x reference)
import numpy as np
import jax
import jax.numpy as jnp
from jax import lax
from jax.experimental import pallas as pl
from jax.experimental.pallas import tpu as pltpu

EMBED = 20; HID = 20; HID2 = 20
NUM_SYMBOLS = 1000000; NUM_ACTIONS = 1000; SEQ = 8192; B = 1
MEM_H = 16; MEM_W = 16; N_READ = 4; CTRL = 64
CLIP = 20.0; EPS = 1e-6
HP = lax.Precision.HIGHEST

def _eye(n):
    r = lax.broadcasted_iota(jnp.int32, (n, n), 0)
    c = lax.broadcasted_iota(jnp.int32, (n, n), 1)
    return jnp.where(r == c, jnp.float32(1.0), jnp.float32(0.0))


def _tpose(a, n):
    return lax.dot_general(_eye(n), a, (((1,), (1,)), ((), ())), precision=HP)


def _dnc_kernel(idx_ref,
                emb_ref,
                mask_ref,
                wihT_ref,
                whhT_ref,
                bih_ref, bhh_ref,
                cwihT_ref,
                cbih_ref, cbhh_ref,
                wheadsT_ref,
                bheads_ref,
                out_whT_ref,
                out_wrT_ref,
                out_b_ref,
                lin_wT_ref,
                lin_b_ref,
                act_wT_ref,
                act_b_ref,
                out_ref,
                x2_ref,
                gi_ref, gf_ref, gg_ref, go_ref,
                sem):
    def issue(t, carry):
        pltpu.make_async_copy(
            emb_ref.at[pl.ds(idx_ref[t], 1), :],
            x2_ref.at[pl.ds(t, 1), :],
            sem).start()
        return carry
    lax.fori_loop(0, SEQ, issue, 0)
    pltpu.make_async_copy(emb_ref.at[pl.ds(0, SEQ), :], x2_ref, sem).wait()

    x2_ref[...] = x2_ref[...] * mask_ref[...]
    x2 = x2_ref[...]
    b = bih_ref[...] + bhh_ref[...]
    gi_ref[...] = jnp.dot(x2, wihT_ref[:, 0:HID], preferred_element_type=jnp.float32) + b[:, 0:HID]
    gf_ref[...] = jnp.dot(x2, wihT_ref[:, HID:2*HID], preferred_element_type=jnp.float32) + b[:, HID:2*HID]
    gg_ref[...] = jnp.dot(x2, wihT_ref[:, 2*HID:3*HID], preferred_element_type=jnp.float32) + b[:, 2*HID:3*HID]
    go_ref[...] = jnp.dot(x2, wihT_ref[:, 3*HID:4*HID], preferred_element_type=jnp.float32) + b[:, 3*HID:4*HID]

    whT_i = whhT_ref[:, 0:HID]
    whT_f = whhT_ref[:, HID:2*HID]
    whT_g = whhT_ref[:, 2*HID:3*HID]
    whT_o = whhT_ref[:, 3*HID:4*HID]

    def step_chunk(it, carry):
        h, c, s = carry
        o = it * 8
        ci = gi_ref[pl.ds(o, 8), :]
        cf = gf_ref[pl.ds(o, 8), :]
        cg = gg_ref[pl.ds(o, 8), :]
        co = go_ref[pl.ds(o, 8), :]
        for r in range(8):
            hi = jnp.dot(h, whT_i, preferred_element_type=jnp.float32)
            hf = jnp.dot(h, whT_f, preferred_element_type=jnp.float32)
            hg = jnp.dot(h, whT_g, preferred_element_type=jnp.float32)
            ho = jnp.dot(h, whT_o, preferred_element_type=jnp.float32)
            ii = jax.nn.sigmoid(ci[r:r+1, :] + hi)
            ff = jax.nn.sigmoid(cf[r:r+1, :] + hf)
            tg = jnp.tanh(cg[r:r+1, :] + hg)
            oo = jax.nn.sigmoid(co[r:r+1, :] + ho)
            c = ff * c + ii * tg
            h = oo * jnp.tanh(c)
            s = s + h
        return (h, c, s)

    z = jnp.zeros((1, HID), jnp.float32)
    _, _, x4 = lax.fori_loop(0, SEQ // 8, step_chunk, (z, z, z))

    gc = jnp.dot(x4, cwihT_ref[...], preferred_element_type=jnp.float32, precision=HP) \
        + cbih_ref[...] + cbhh_ref[...]
    cc = jax.nn.sigmoid(gc[:, 0:CTRL]) * jnp.tanh(gc[:, 2*CTRL:3*CTRL])
    hc = jax.nn.sigmoid(gc[:, 3*CTRL:4*CTRL]) * jnp.tanh(cc)
    hc = jnp.clip(hc, -CLIP, CLIP)

    P = jnp.dot(hc, wheadsT_ref[...], preferred_element_type=jnp.float32, precision=HP) \
        + bheads_ref[...]
    wg = jax.nn.sigmoid(P[:, 0:1])
    wa = jax.nn.sigmoid(P[:, 1:2])
    erase = jax.nn.sigmoid(P[:, 2:18])
    add = jnp.tanh(P[:, 18:34])

    pw = lax.broadcasted_iota(jnp.int32, (1, MEM_H), 1).astype(jnp.float32)
    alloc = jnp.float32(1.0 - EPS) * jnp.exp(pw * jnp.log(jnp.float32(EPS)))
    wl_w = wg * (wa * alloc + (1.0 - wa) * jnp.float32(1.0 / MEM_H))

    wlw_col = _tpose(wl_w, MEM_H)
    mem = jnp.float32(1e-6) * (1.0 - wlw_col * erase) + wlw_col * add

    rkey = jnp.concatenate(
        [P[:, 34 + MEM_W * i: 34 + MEM_W * (i + 1)] for i in range(N_READ)],
        axis=0)
    rkey = jnp.tanh(rkey)
    rbeta = jnp.concatenate(
        [P[:, 98 + i: 99 + i] for i in range(N_READ)], axis=0)
    rbeta = jax.nn.softplus(rbeta)
    modes = jnp.concatenate(
        [P[:, 102 + 3 * i: 102 + 3 * (i + 1)] for i in range(N_READ)],
        axis=0)
    modes = jax.nn.softmax(modes, axis=-1)

    kn = rkey / (jnp.sqrt(jnp.sum(rkey * rkey, axis=-1, keepdims=True)) + EPS)
    mn = mem / (jnp.sqrt(jnp.sum(mem * mem, axis=-1, keepdims=True)) + EPS)
    scores = lax.dot_general(kn, mn, (((1,), (1,)), ((), ())), precision=HP)
    wc_r = jax.nn.softmax(rbeta * scores, axis=-1)
    wl_r = modes[:, 2:3] * wc_r
    rv4 = jnp.dot(wl_r, mem, preferred_element_type=jnp.float32, precision=HP)
    rv = jnp.concatenate([rv4[i:i+1, :] for i in range(N_READ)], axis=1)

    x4b = jnp.dot(hc, out_whT_ref[...], preferred_element_type=jnp.float32, precision=HP) \
        + jnp.dot(rv, out_wrT_ref[...], preferred_element_type=jnp.float32, precision=HP) \
        + out_b_ref[...]

    x4c = jnp.concatenate([x4, x4b], axis=1)
    x5 = jax.nn.relu(
        jnp.dot(x4c, lin_wT_ref[...], preferred_element_type=jnp.float32, precision=HP)
        + lin_b_ref[...])
    out_ref[...] = jnp.dot(x5, act_wT_ref[...], preferred_element_type=jnp.float32,
                           precision=HP) + act_b_ref[...]


def kernel(x, emb, lstm_Wih, lstm_Whh, lstm_bih, lstm_bhh, ctrl_Wih, ctrl_Whh,
           ctrl_bih, ctrl_bhh, w_key_W, w_key_b, w_beta_W, w_beta_b, w_alloc_W,
           w_alloc_b, w_gate_W, w_gate_b, w_erase_W, w_erase_b, w_add_W,
           w_add_b, r_key_W, r_key_b, r_beta_W, r_beta_b, r_free_W, r_free_b,
           r_mode_W, r_mode_b, out_W, out_b, lin_W, lin_b, act_W, act_b):
    idx = x[0].astype(jnp.int32)
    mask = (idx != NUM_SYMBOLS).astype(jnp.float32)[:, None]
    mask20 = jnp.broadcast_to(mask, (SEQ, 32))

    heads_W = jnp.concatenate([w_gate_W, w_alloc_W, w_erase_W, w_add_W,
                               r_key_W, r_beta_W, r_mode_W], axis=0)
    heads_b = jnp.concatenate([w_gate_b, w_alloc_b, w_erase_b, w_add_b,
                               r_key_b, r_beta_b, r_mode_b], axis=0)

    embp = jnp.pad(emb, ((0, 0), (0, 32 - EMBED)))
    wih_p = jnp.pad(lstm_Wih, ((0, 0), (0, 32 - EMBED)))
    args = (
        embp,
        mask20,
        wih_p.T,
        lstm_Whh.T,
        lstm_bih.reshape(1, -1), lstm_bhh.reshape(1, -1),
        ctrl_Wih[:, :HID].T,
        ctrl_bih.reshape(1, -1), ctrl_bhh.reshape(1, -1),
        heads_W.T,
        heads_b.reshape(1, -1),
        out_W[:, :CTRL].T,
        out_W[:, CTRL:].T,
        out_b.reshape(1, -1),
        lin_W.T,
        lin_b.reshape(1, -1),
        act_W.T,
        act_b.reshape(1, -1),
    )

    grid_spec = pltpu.PrefetchScalarGridSpec(
        num_scalar_prefetch=1,
        grid=(1,),
        in_specs=[
            pl.BlockSpec(memory_space=pl.ANY),
        ] + [pl.BlockSpec(memory_space=pltpu.VMEM)] * (len(args) - 1),
        out_specs=pl.BlockSpec(memory_space=pltpu.VMEM),
        scratch_shapes=[
            pltpu.VMEM((SEQ, 32), jnp.float32),
            pltpu.VMEM((SEQ, HID), jnp.float32),
            pltpu.VMEM((SEQ, HID), jnp.float32),
            pltpu.VMEM((SEQ, HID), jnp.float32),
            pltpu.VMEM((SEQ, HID), jnp.float32),
            pltpu.SemaphoreType.DMA,
        ],
    )

    return pl.pallas_call(
        _dnc_kernel,
        grid_spec=grid_spec,
        out_shape=jax.ShapeDtypeStruct((1, NUM_ACTIONS), jnp.float32),
        compiler_params=pltpu.CompilerParams(
            dimension_semantics=("arbitrary",),
            vmem_limit_bytes=60 * 1024 * 1024,
            disable_bounds_checks=True,
        ),
    )(idx, *args)


# speedup vs baseline: 1.2948x; 1.2948x over previous
import numpy as np
import jax
import jax.numpy as jnp
from jax import lax
from jax.experimental import pallas as pl
from jax.experimental.pallas import tpu as pltpu

EMBED = 20; HID = 20; HID2 = 20
NUM_SYMBOLS = 1000000; NUM_ACTIONS = 1000; SEQ = 8192; B = 1
MEM_H = 16; MEM_W = 16; N_READ = 4; CTRL = 64
CLIP = 20.0; EPS = 1e-6
HP = lax.Precision.HIGHEST

def _eye(n):
    r = lax.broadcasted_iota(jnp.int32, (n, n), 0)
    c = lax.broadcasted_iota(jnp.int32, (n, n), 1)
    return jnp.where(r == c, jnp.float32(1.0), jnp.float32(0.0))


def _tpose(a, n):
    return lax.dot_general(_eye(n), a, (((1,), (1,)), ((), ())), precision=HP)


def _dnc_kernel(idx_ref,
                emb_ref,
                mask_ref,
                wihT_ref,
                whhT_ref,
                bih_ref, bhh_ref,
                cwihT_ref,
                cbih_ref, cbhh_ref,
                wheadsT_ref,
                bheads_ref,
                out_whT_ref,
                out_wrT_ref,
                out_b_ref,
                lin_wT_ref,
                lin_b_ref,
                act_wT_ref,
                act_b_ref,
                out_ref,
                x2_ref,
                g_ref,
                sem):
    def issue(t, carry):
        pltpu.make_async_copy(
            emb_ref.at[pl.ds(idx_ref[t], 1), :],
            x2_ref.at[pl.ds(t, 1), :],
            sem).start()
        return carry
    lax.fori_loop(0, SEQ, issue, 0)
    pltpu.make_async_copy(emb_ref.at[pl.ds(0, SEQ), :], x2_ref, sem).wait()

    x2_ref[...] = x2_ref[...] * mask_ref[...]
    x2 = x2_ref[...]
    b = bih_ref[...] + bhh_ref[...]
    g_ref[...] = jnp.dot(x2, wihT_ref[...], preferred_element_type=jnp.float32) + b

    whT = whhT_ref[...]

    def step_chunk(it, carry):
        h, c, s = carry
        o = it * 8
        ch = g_ref[pl.ds(o, 8), :]
        for r in range(8):
            g = ch[r:r+1, :] + jnp.dot(h, whT, preferred_element_type=jnp.float32)
            ii = jax.nn.sigmoid(g[:, 0:HID])
            ff = jax.nn.sigmoid(g[:, 128:128+HID])
            tg = jnp.tanh(g[:, 256:256+HID])
            oo = jax.nn.sigmoid(g[:, 384:384+HID])
            c = ff * c + ii * tg
            h = oo * jnp.tanh(c)
            s = s + h
        return (h, c, s)

    z = jnp.zeros((1, HID), jnp.float32)
    _, _, x4 = lax.fori_loop(0, SEQ // 8, step_chunk, (z, z, z))

    gc = jnp.dot(x4, cwihT_ref[...], preferred_element_type=jnp.float32, precision=HP) \
        + cbih_ref[...] + cbhh_ref[...]
    cc = jax.nn.sigmoid(gc[:, 0:CTRL]) * jnp.tanh(gc[:, 2*CTRL:3*CTRL])
    hc = jax.nn.sigmoid(gc[:, 3*CTRL:4*CTRL]) * jnp.tanh(cc)
    hc = jnp.clip(hc, -CLIP, CLIP)

    P = jnp.dot(hc, wheadsT_ref[...], preferred_element_type=jnp.float32, precision=HP) \
        + bheads_ref[...]
    wg = jax.nn.sigmoid(P[:, 0:1])
    wa = jax.nn.sigmoid(P[:, 1:2])
    erase = jax.nn.sigmoid(P[:, 2:18])
    add = jnp.tanh(P[:, 18:34])

    pw = lax.broadcasted_iota(jnp.int32, (1, MEM_H), 1).astype(jnp.float32)
    alloc = jnp.float32(1.0 - EPS) * jnp.exp(pw * jnp.log(jnp.float32(EPS)))
    wl_w = wg * (wa * alloc + (1.0 - wa) * jnp.float32(1.0 / MEM_H))

    wlw_col = _tpose(wl_w, MEM_H)
    mem = jnp.float32(1e-6) * (1.0 - wlw_col * erase) + wlw_col * add

    rkey = jnp.concatenate(
        [P[:, 34 + MEM_W * i: 34 + MEM_W * (i + 1)] for i in range(N_READ)],
        axis=0)
    rkey = jnp.tanh(rkey)
    rbeta = jnp.concatenate(
        [P[:, 98 + i: 99 + i] for i in range(N_READ)], axis=0)
    rbeta = jax.nn.softplus(rbeta)
    modes = jnp.concatenate(
        [P[:, 102 + 3 * i: 102 + 3 * (i + 1)] for i in range(N_READ)],
        axis=0)
    modes = jax.nn.softmax(modes, axis=-1)

    kn = rkey / (jnp.sqrt(jnp.sum(rkey * rkey, axis=-1, keepdims=True)) + EPS)
    mn = mem / (jnp.sqrt(jnp.sum(mem * mem, axis=-1, keepdims=True)) + EPS)
    scores = lax.dot_general(kn, mn, (((1,), (1,)), ((), ())), precision=HP)
    wc_r = jax.nn.softmax(rbeta * scores, axis=-1)
    wl_r = modes[:, 2:3] * wc_r
    rv4 = jnp.dot(wl_r, mem, preferred_element_type=jnp.float32, precision=HP)
    rv = jnp.concatenate([rv4[i:i+1, :] for i in range(N_READ)], axis=1)

    x4b = jnp.dot(hc, out_whT_ref[...], preferred_element_type=jnp.float32, precision=HP) \
        + jnp.dot(rv, out_wrT_ref[...], preferred_element_type=jnp.float32, precision=HP) \
        + out_b_ref[...]

    x4c = jnp.concatenate([x4, x4b], axis=1)
    x5 = jax.nn.relu(
        jnp.dot(x4c, lin_wT_ref[...], preferred_element_type=jnp.float32, precision=HP)
        + lin_b_ref[...])
    out_ref[...] = jnp.dot(x5, act_wT_ref[...], preferred_element_type=jnp.float32,
                           precision=HP) + act_b_ref[...]


def kernel(x, emb, lstm_Wih, lstm_Whh, lstm_bih, lstm_bhh, ctrl_Wih, ctrl_Whh,
           ctrl_bih, ctrl_bhh, w_key_W, w_key_b, w_beta_W, w_beta_b, w_alloc_W,
           w_alloc_b, w_gate_W, w_gate_b, w_erase_W, w_erase_b, w_add_W,
           w_add_b, r_key_W, r_key_b, r_beta_W, r_beta_b, r_free_W, r_free_b,
           r_mode_W, r_mode_b, out_W, out_b, lin_W, lin_b, act_W, act_b):
    idx = x[0].astype(jnp.int32)
    mask = (idx != NUM_SYMBOLS).astype(jnp.float32)[:, None]
    mask20 = jnp.broadcast_to(mask, (SEQ, EMBED))

    heads_W = jnp.concatenate([w_gate_W, w_alloc_W, w_erase_W, w_add_W,
                               r_key_W, r_beta_W, r_mode_W], axis=0)
    heads_b = jnp.concatenate([w_gate_b, w_alloc_b, w_erase_b, w_add_b,
                               r_key_b, r_beta_b, r_mode_b], axis=0)

    def _fields(w):
        wt = w.T
        z = jnp.zeros((wt.shape[0], 128 - HID), wt.dtype)
        return jnp.concatenate([wt[:, 0:HID], z, wt[:, HID:2*HID], z,
                                wt[:, 2*HID:3*HID], z, wt[:, 3*HID:4*HID], z], axis=1)
    args = (
        emb,
        mask20,
        _fields(lstm_Wih),
        _fields(lstm_Whh),
        _fields(lstm_bih.reshape(-1, 1)),
        _fields(lstm_bhh.reshape(-1, 1)),
        ctrl_Wih[:, :HID].T,
        ctrl_bih.reshape(1, -1), ctrl_bhh.reshape(1, -1),
        heads_W.T,
        heads_b.reshape(1, -1),
        out_W[:, :CTRL].T,
        out_W[:, CTRL:].T,
        out_b.reshape(1, -1),
        lin_W.T,
        lin_b.reshape(1, -1),
        act_W.T,
        act_b.reshape(1, -1),
    )

    grid_spec = pltpu.PrefetchScalarGridSpec(
        num_scalar_prefetch=1,
        grid=(1,),
        in_specs=[
            pl.BlockSpec(memory_space=pl.ANY),
        ] + [pl.BlockSpec(memory_space=pltpu.VMEM)] * (len(args) - 1),
        out_specs=pl.BlockSpec(memory_space=pltpu.VMEM),
        scratch_shapes=[
            pltpu.VMEM((SEQ, EMBED), jnp.float32),
            pltpu.VMEM((SEQ, 512), jnp.float32),
            pltpu.SemaphoreType.DMA,
        ],
    )

    return pl.pallas_call(
        _dnc_kernel,
        grid_spec=grid_spec,
        out_shape=jax.ShapeDtypeStruct((1, NUM_ACTIONS), jnp.float32),
        compiler_params=pltpu.CompilerParams(
            dimension_semantics=("arbitrary",),
            vmem_limit_bytes=60 * 1024 * 1024,
            disable_bounds_checks=True,
        ),
    )(idx, *args)


# speedup vs baseline: 1.3290x; 1.0265x over previous
import numpy as np
import jax
import jax.numpy as jnp
from jax import lax
from jax.experimental import pallas as pl
from jax.experimental.pallas import tpu as pltpu

EMBED = 20; HID = 20; HID2 = 20
NUM_SYMBOLS = 1000000; NUM_ACTIONS = 1000; SEQ = 8192; B = 1
MEM_H = 16; MEM_W = 16; N_READ = 4; CTRL = 64
CLIP = 20.0; EPS = 1e-6
HP = lax.Precision.HIGHEST

def _eye(n):
    r = lax.broadcasted_iota(jnp.int32, (n, n), 0)
    c = lax.broadcasted_iota(jnp.int32, (n, n), 1)
    return jnp.where(r == c, jnp.float32(1.0), jnp.float32(0.0))


def _tpose(a, n):
    return lax.dot_general(_eye(n), a, (((1,), (1,)), ((), ())), precision=HP)


def _dnc_kernel(idx_ref,
                emb_ref,
                mask_ref,
                wihT_ref,
                whhT_ref,
                bih_ref, bhh_ref,
                cwihT_ref,
                cbih_ref, cbhh_ref,
                wheadsT_ref,
                bheads_ref,
                out_whT_ref,
                out_wrT_ref,
                out_b_ref,
                lin_wT_ref,
                lin_b_ref,
                act_wT_ref,
                act_b_ref,
                out_ref,
                x2_ref,
                g_ref,
                sem):
    def issue(t, carry):
        pltpu.make_async_copy(
            emb_ref.at[pl.ds(idx_ref[t], 1), :],
            x2_ref.at[pl.ds(t, 1), :],
            sem).start()
        return carry
    lax.fori_loop(0, SEQ, issue, 0)
    pltpu.make_async_copy(emb_ref.at[pl.ds(0, SEQ), :], x2_ref, sem).wait()

    x2_ref[...] = x2_ref[...] * mask_ref[...]
    x2 = x2_ref[...]
    b = bih_ref[...] + bhh_ref[...]
    g_ref[...] = jnp.dot(x2, wihT_ref[...], preferred_element_type=jnp.float32) + b

    whT = whhT_ref[...]

    def step_chunk(it, carry):
        h, c, s = carry
        o = it * 8
        ch = g_ref[pl.ds(o, 8), :]
        for r in range(8):
            g = ch[r:r+1, :] + jnp.dot(h, whT, preferred_element_type=jnp.float32)
            ii = 0.5 * jnp.tanh(0.5 * g[:, 0:HID]) + 0.5
            ff = 0.5 * jnp.tanh(0.5 * g[:, 128:128+HID]) + 0.5
            tg = jnp.tanh(g[:, 256:256+HID])
            oo = 0.5 * jnp.tanh(0.5 * g[:, 384:384+HID]) + 0.5
            c = ff * c + ii * tg
            h = oo * jnp.tanh(c)
            s = s + h
        return (h, c, s)

    z = jnp.zeros((1, HID), jnp.float32)
    _, _, x4 = lax.fori_loop(0, SEQ // 8, step_chunk, (z, z, z))

    gc = jnp.dot(x4, cwihT_ref[...], preferred_element_type=jnp.float32, precision=HP) \
        + cbih_ref[...] + cbhh_ref[...]
    cc = jax.nn.sigmoid(gc[:, 0:CTRL]) * jnp.tanh(gc[:, 2*CTRL:3*CTRL])
    hc = jax.nn.sigmoid(gc[:, 3*CTRL:4*CTRL]) * jnp.tanh(cc)
    hc = jnp.clip(hc, -CLIP, CLIP)

    P = jnp.dot(hc, wheadsT_ref[...], preferred_element_type=jnp.float32, precision=HP) \
        + bheads_ref[...]
    wg = jax.nn.sigmoid(P[:, 0:1])
    wa = jax.nn.sigmoid(P[:, 1:2])
    erase = jax.nn.sigmoid(P[:, 2:18])
    add = jnp.tanh(P[:, 18:34])

    pw = lax.broadcasted_iota(jnp.int32, (1, MEM_H), 1).astype(jnp.float32)
    alloc = jnp.float32(1.0 - EPS) * jnp.exp(pw * jnp.log(jnp.float32(EPS)))
    wl_w = wg * (wa * alloc + (1.0 - wa) * jnp.float32(1.0 / MEM_H))

    wlw_col = _tpose(wl_w, MEM_H)
    mem = jnp.float32(1e-6) * (1.0 - wlw_col * erase) + wlw_col * add

    rkey = jnp.concatenate(
        [P[:, 34 + MEM_W * i: 34 + MEM_W * (i + 1)] for i in range(N_READ)],
        axis=0)
    rkey = jnp.tanh(rkey)
    rbeta = jnp.concatenate(
        [P[:, 98 + i: 99 + i] for i in range(N_READ)], axis=0)
    rbeta = jax.nn.softplus(rbeta)
    modes = jnp.concatenate(
        [P[:, 102 + 3 * i: 102 + 3 * (i + 1)] for i in range(N_READ)],
        axis=0)
    modes = jax.nn.softmax(modes, axis=-1)

    kn = rkey / (jnp.sqrt(jnp.sum(rkey * rkey, axis=-1, keepdims=True)) + EPS)
    mn = mem / (jnp.sqrt(jnp.sum(mem * mem, axis=-1, keepdims=True)) + EPS)
    scores = lax.dot_general(kn, mn, (((1,), (1,)), ((), ())), precision=HP)
    wc_r = jax.nn.softmax(rbeta * scores, axis=-1)
    wl_r = modes[:, 2:3] * wc_r
    rv4 = jnp.dot(wl_r, mem, preferred_element_type=jnp.float32, precision=HP)
    rv = jnp.concatenate([rv4[i:i+1, :] for i in range(N_READ)], axis=1)

    x4b = jnp.dot(hc, out_whT_ref[...], preferred_element_type=jnp.float32, precision=HP) \
        + jnp.dot(rv, out_wrT_ref[...], preferred_element_type=jnp.float32, precision=HP) \
        + out_b_ref[...]

    x4c = jnp.concatenate([x4, x4b], axis=1)
    x5 = jax.nn.relu(
        jnp.dot(x4c, lin_wT_ref[...], preferred_element_type=jnp.float32, precision=HP)
        + lin_b_ref[...])
    out_ref[...] = jnp.dot(x5, act_wT_ref[...], preferred_element_type=jnp.float32,
                           precision=HP) + act_b_ref[...]


def kernel(x, emb, lstm_Wih, lstm_Whh, lstm_bih, lstm_bhh, ctrl_Wih, ctrl_Whh,
           ctrl_bih, ctrl_bhh, w_key_W, w_key_b, w_beta_W, w_beta_b, w_alloc_W,
           w_alloc_b, w_gate_W, w_gate_b, w_erase_W, w_erase_b, w_add_W,
           w_add_b, r_key_W, r_key_b, r_beta_W, r_beta_b, r_free_W, r_free_b,
           r_mode_W, r_mode_b, out_W, out_b, lin_W, lin_b, act_W, act_b):
    idx = x[0].astype(jnp.int32)
    mask = (idx != NUM_SYMBOLS).astype(jnp.float32)[:, None]
    mask20 = jnp.broadcast_to(mask, (SEQ, EMBED))

    heads_W = jnp.concatenate([w_gate_W, w_alloc_W, w_erase_W, w_add_W,
                               r_key_W, r_beta_W, r_mode_W], axis=0)
    heads_b = jnp.concatenate([w_gate_b, w_alloc_b, w_erase_b, w_add_b,
                               r_key_b, r_beta_b, r_mode_b], axis=0)

    def _fields(w):
        wt = w.T
        z = jnp.zeros((wt.shape[0], 128 - HID), wt.dtype)
        return jnp.concatenate([wt[:, 0:HID], z, wt[:, HID:2*HID], z,
                                wt[:, 2*HID:3*HID], z, wt[:, 3*HID:4*HID], z], axis=1)
    args = (
        emb,
        mask20,
        _fields(lstm_Wih),
        _fields(lstm_Whh),
        _fields(lstm_bih.reshape(-1, 1)),
        _fields(lstm_bhh.reshape(-1, 1)),
        ctrl_Wih[:, :HID].T,
        ctrl_bih.reshape(1, -1), ctrl_bhh.reshape(1, -1),
        heads_W.T,
        heads_b.reshape(1, -1),
        out_W[:, :CTRL].T,
        out_W[:, CTRL:].T,
        out_b.reshape(1, -1),
        lin_W.T,
        lin_b.reshape(1, -1),
        act_W.T,
        act_b.reshape(1, -1),
    )

    grid_spec = pltpu.PrefetchScalarGridSpec(
        num_scalar_prefetch=1,
        grid=(1,),
        in_specs=[
            pl.BlockSpec(memory_space=pl.ANY),
        ] + [pl.BlockSpec(memory_space=pltpu.VMEM)] * (len(args) - 1),
        out_specs=pl.BlockSpec(memory_space=pltpu.VMEM),
        scratch_shapes=[
            pltpu.VMEM((SEQ, EMBED), jnp.float32),
            pltpu.VMEM((SEQ, 512), jnp.float32),
            pltpu.SemaphoreType.DMA,
        ],
    )

    return pl.pallas_call(
        _dnc_kernel,
        grid_spec=grid_spec,
        out_shape=jax.ShapeDtypeStruct((1, NUM_ACTIONS), jnp.float32),
        compiler_params=pltpu.CompilerParams(
            dimension_semantics=("arbitrary",),
            vmem_limit_bytes=60 * 1024 * 1024,
            disable_bounds_checks=True,
        ),
    )(idx, *args)


# speedup vs baseline: 1.3468x; 1.0134x over previous
import numpy as np
import jax
import jax.numpy as jnp
from jax import lax
from jax.experimental import pallas as pl
from jax.experimental.pallas import tpu as pltpu

EMBED = 20; HID = 20; HID2 = 20
NUM_SYMBOLS = 1000000; NUM_ACTIONS = 1000; SEQ = 8192; B = 1
MEM_H = 16; MEM_W = 16; N_READ = 4; CTRL = 64
CLIP = 20.0; EPS = 1e-6
HP = lax.Precision.HIGHEST

def _eye(n):
    r = lax.broadcasted_iota(jnp.int32, (n, n), 0)
    c = lax.broadcasted_iota(jnp.int32, (n, n), 1)
    return jnp.where(r == c, jnp.float32(1.0), jnp.float32(0.0))


def _tpose(a, n):
    return lax.dot_general(_eye(n), a, (((1,), (1,)), ((), ())), precision=HP)


def _dnc_kernel(idx_ref,
                emb_ref,
                mask_ref,
                wihT_ref,
                whhT_ref,
                bih_ref, bhh_ref,
                cwihT_ref,
                cbih_ref, cbhh_ref,
                wheadsT_ref,
                bheads_ref,
                out_whT_ref,
                out_wrT_ref,
                out_b_ref,
                lin_wT_ref,
                lin_b_ref,
                act_wT_ref,
                act_b_ref,
                out_ref,
                x2_ref,
                g_ref,
                sem):
    def issue(tt, carry):
        base = tt * 8
        for k in range(8):
            t = base + k
            pltpu.make_async_copy(
                emb_ref.at[pl.ds(idx_ref[t], 1), :],
                x2_ref.at[pl.ds(t, 1), :],
                sem).start()
        return carry
    lax.fori_loop(0, SEQ // 8, issue, 0)
    pltpu.make_async_copy(emb_ref.at[pl.ds(0, SEQ), :], x2_ref, sem).wait()

    x2_ref[...] = x2_ref[...] * mask_ref[...]
    x2 = x2_ref[...]
    b = bih_ref[...] + bhh_ref[...]
    g_ref[...] = jnp.dot(x2, wihT_ref[...], preferred_element_type=jnp.float32) + b

    whT = whhT_ref[...]

    def step_chunk(it, carry):
        h, c, s = carry
        o = it * 8
        ch = g_ref[pl.ds(o, 8), :]
        for r in range(8):
            g = ch[r:r+1, :] + jnp.dot(h, whT, preferred_element_type=jnp.float32)
            ii = 0.5 * jnp.tanh(0.5 * g[:, 0:HID]) + 0.5
            ff = 0.5 * jnp.tanh(0.5 * g[:, 128:128+HID]) + 0.5
            tg = jnp.tanh(g[:, 256:256+HID])
            oo = 0.5 * jnp.tanh(0.5 * g[:, 384:384+HID]) + 0.5
            c = ff * c + ii * tg
            h = oo * jnp.tanh(c)
            s = s + h
        return (h, c, s)

    z = jnp.zeros((1, HID), jnp.float32)
    _, _, x4 = lax.fori_loop(0, SEQ // 8, step_chunk, (z, z, z))

    gc = jnp.dot(x4, cwihT_ref[...], preferred_element_type=jnp.float32, precision=HP) \
        + cbih_ref[...] + cbhh_ref[...]
    cc = jax.nn.sigmoid(gc[:, 0:CTRL]) * jnp.tanh(gc[:, 2*CTRL:3*CTRL])
    hc = jax.nn.sigmoid(gc[:, 3*CTRL:4*CTRL]) * jnp.tanh(cc)
    hc = jnp.clip(hc, -CLIP, CLIP)

    P = jnp.dot(hc, wheadsT_ref[...], preferred_element_type=jnp.float32, precision=HP) \
        + bheads_ref[...]
    wg = jax.nn.sigmoid(P[:, 0:1])
    wa = jax.nn.sigmoid(P[:, 1:2])
    erase = jax.nn.sigmoid(P[:, 2:18])
    add = jnp.tanh(P[:, 18:34])

    pw = lax.broadcasted_iota(jnp.int32, (1, MEM_H), 1).astype(jnp.float32)
    alloc = jnp.float32(1.0 - EPS) * jnp.exp(pw * jnp.log(jnp.float32(EPS)))
    wl_w = wg * (wa * alloc + (1.0 - wa) * jnp.float32(1.0 / MEM_H))

    wlw_col = _tpose(wl_w, MEM_H)
    mem = jnp.float32(1e-6) * (1.0 - wlw_col * erase) + wlw_col * add

    rkey = jnp.concatenate(
        [P[:, 34 + MEM_W * i: 34 + MEM_W * (i + 1)] for i in range(N_READ)],
        axis=0)
    rkey = jnp.tanh(rkey)
    rbeta = jnp.concatenate(
        [P[:, 98 + i: 99 + i] for i in range(N_READ)], axis=0)
    rbeta = jax.nn.softplus(rbeta)
    modes = jnp.concatenate(
        [P[:, 102 + 3 * i: 102 + 3 * (i + 1)] for i in range(N_READ)],
        axis=0)
    modes = jax.nn.softmax(modes, axis=-1)

    kn = rkey / (jnp.sqrt(jnp.sum(rkey * rkey, axis=-1, keepdims=True)) + EPS)
    mn = mem / (jnp.sqrt(jnp.sum(mem * mem, axis=-1, keepdims=True)) + EPS)
    scores = lax.dot_general(kn, mn, (((1,), (1,)), ((), ())), precision=HP)
    wc_r = jax.nn.softmax(rbeta * scores, axis=-1)
    wl_r = modes[:, 2:3] * wc_r
    rv4 = jnp.dot(wl_r, mem, preferred_element_type=jnp.float32, precision=HP)
    rv = jnp.concatenate([rv4[i:i+1, :] for i in range(N_READ)], axis=1)

    x4b = jnp.dot(hc, out_whT_ref[...], preferred_element_type=jnp.float32, precision=HP) \
        + jnp.dot(rv, out_wrT_ref[...], preferred_element_type=jnp.float32, precision=HP) \
        + out_b_ref[...]

    x4c = jnp.concatenate([x4, x4b], axis=1)
    x5 = jax.nn.relu(
        jnp.dot(x4c, lin_wT_ref[...], preferred_element_type=jnp.float32, precision=HP)
        + lin_b_ref[...])
    out_ref[...] = jnp.dot(x5, act_wT_ref[...], preferred_element_type=jnp.float32,
                           precision=HP) + act_b_ref[...]


def kernel(x, emb, lstm_Wih, lstm_Whh, lstm_bih, lstm_bhh, ctrl_Wih, ctrl_Whh,
           ctrl_bih, ctrl_bhh, w_key_W, w_key_b, w_beta_W, w_beta_b, w_alloc_W,
           w_alloc_b, w_gate_W, w_gate_b, w_erase_W, w_erase_b, w_add_W,
           w_add_b, r_key_W, r_key_b, r_beta_W, r_beta_b, r_free_W, r_free_b,
           r_mode_W, r_mode_b, out_W, out_b, lin_W, lin_b, act_W, act_b):
    idx = x[0].astype(jnp.int32)
    mask = (idx != NUM_SYMBOLS).astype(jnp.float32)[:, None]
    mask20 = jnp.broadcast_to(mask, (SEQ, EMBED))

    heads_W = jnp.concatenate([w_gate_W, w_alloc_W, w_erase_W, w_add_W,
                               r_key_W, r_beta_W, r_mode_W], axis=0)
    heads_b = jnp.concatenate([w_gate_b, w_alloc_b, w_erase_b, w_add_b,
                               r_key_b, r_beta_b, r_mode_b], axis=0)

    def _fields(w):
        wt = w.T
        z = jnp.zeros((wt.shape[0], 128 - HID), wt.dtype)
        return jnp.concatenate([wt[:, 0:HID], z, wt[:, HID:2*HID], z,
                                wt[:, 2*HID:3*HID], z, wt[:, 3*HID:4*HID], z], axis=1)
    args = (
        emb,
        mask20,
        _fields(lstm_Wih),
        _fields(lstm_Whh),
        _fields(lstm_bih.reshape(-1, 1)),
        _fields(lstm_bhh.reshape(-1, 1)),
        ctrl_Wih[:, :HID].T,
        ctrl_bih.reshape(1, -1), ctrl_bhh.reshape(1, -1),
        heads_W.T,
        heads_b.reshape(1, -1),
        out_W[:, :CTRL].T,
        out_W[:, CTRL:].T,
        out_b.reshape(1, -1),
        lin_W.T,
        lin_b.reshape(1, -1),
        act_W.T,
        act_b.reshape(1, -1),
    )

    grid_spec = pltpu.PrefetchScalarGridSpec(
        num_scalar_prefetch=1,
        grid=(1,),
        in_specs=[
            pl.BlockSpec(memory_space=pl.ANY),
        ] + [pl.BlockSpec(memory_space=pltpu.VMEM)] * (len(args) - 1),
        out_specs=pl.BlockSpec(memory_space=pltpu.VMEM),
        scratch_shapes=[
            pltpu.VMEM((SEQ, EMBED), jnp.float32),
            pltpu.VMEM((SEQ, 512), jnp.float32),
            pltpu.SemaphoreType.DMA,
        ],
    )

    return pl.pallas_call(
        _dnc_kernel,
        grid_spec=grid_spec,
        out_shape=jax.ShapeDtypeStruct((1, NUM_ACTIONS), jnp.float32),
        compiler_params=pltpu.CompilerParams(
            dimension_semantics=("arbitrary",),
            vmem_limit_bytes=60 * 1024 * 1024,
            disable_bounds_checks=True,
        ),
    )(idx, *args)
